# baseline (speedup 1.0000x reference)
import functools

import jax
import jax.numpy as jnp
from jax import lax
from jax.experimental import pallas as pl
from jax.experimental.pallas import tpu as pltpu
from jax.experimental.pallas import tpu_sc as plsc

_N = 10000
_E = 320000
_D = 128
_DEPTH = 3
_NC = 102

_NCORES = 2
_NSUB = 16
_NW = _NCORES * _NSUB
_C = 128
_EPAD = 327680
_EPW = _EPAD // _NW
_NCH = _EPW // _C


@functools.cache
def _make_segsum(n_pad, yrows, d, nhalf):
    rpt = n_pad // _NSUB
    mesh = plsc.VectorSubcoreMesh(
        core_axis_name="c", subcore_axis_name="s",
        num_cores=_NCORES, num_subcores=_NSUB)

    nbuf = 4

    def body(y1_hbm, y2_hbm, srce_hbm, dste_hbm, zeros_hbm, out_hbm,
             sidx_all, didx_all, rows_v,
             agg_sh, *sems):
        gsems = sems[:nbuf]
        ssems = sems[nbuf:]
        c = lax.axis_index("c")
        s = lax.axis_index("s")
        wid = s * _NCORES + c
        pltpu.sync_copy(srce_hbm.at[pl.ds(wid * _NCH, _NCH)], sidx_all)
        for h in range(nhalf):
            pltpu.sync_copy(
                dste_hbm.at[pl.ds(h * (_EPAD // _C) + wid * _NCH, _NCH)],
                didx_all.at[pl.ds(h * _NCH, _NCH)])

        def one_pass(y_hbm, h, out_base):
            dbase = h * _NCH
            for z0 in range(0, rpt, 128):
                zw = min(128, rpt - z0)
                pltpu.sync_copy(zeros_hbm.at[pl.ds(0, zw)],
                                agg_sh.at[pl.ds(s * rpt + z0, zw)])
            plsc.subcore_barrier()
            for b in range(nbuf):
                pltpu.async_copy(y_hbm.at[sidx_all.at[b]], rows_v.at[b],
                                 gsems[b])

            def step(g, carry):
                for b in range(nbuf):
                    j = g * nbuf + b
                    pltpu.make_async_copy(y_hbm.at[sidx_all.at[j]],
                                          rows_v.at[b], gsems[b]).wait()
                    pltpu.async_copy(rows_v.at[b],
                                     agg_sh.at[didx_all.at[dbase + j]],
                                     ssems[b], add=True)
                for b in range(nbuf):
                    j = g * nbuf + b
                    jn = jnp.minimum(j + nbuf, _NCH - 1)
                    pltpu.make_async_copy(rows_v.at[b],
                                          agg_sh.at[didx_all.at[dbase + jn]],
                                          ssems[b]).wait()
                    pltpu.async_copy(y_hbm.at[sidx_all.at[jn]],
                                     rows_v.at[b], gsems[b])
                return carry

            lax.fori_loop(0, _NCH // nbuf, step, 0)
            for b in range(nbuf):
                pltpu.make_async_copy(y_hbm.at[sidx_all.at[b]],
                                      rows_v.at[b], gsems[b]).wait()
            plsc.subcore_barrier()
            pltpu.sync_copy(
                agg_sh.at[pl.ds(s * rpt, rpt)],
                out_hbm.at[pl.ds(out_base + c * n_pad + s * rpt, rpt)])
            plsc.subcore_barrier()

        for yi, y_hbm in enumerate((y1_hbm, y2_hbm)):
            for h in range(nhalf):
                one_pass(y_hbm, h, (yi * nhalf + h) * 2 * n_pad)

    return pl.kernel(
        body,
        out_type=jax.ShapeDtypeStruct((4 * nhalf * n_pad, d), jnp.float32),
        mesh=mesh,
        scratch_types=[
            pltpu.VMEM((_NCH, _C), jnp.int32),
            pltpu.VMEM((nhalf * _NCH, _C), jnp.int32),
            pltpu.VMEM((nbuf, _C, d), jnp.float32),
            pltpu.VMEM_SHARED((n_pad, d), jnp.float32),
        ] + [pltpu.SemaphoreType.DMA] * (2 * nbuf),
    )


def _segsum_pair(y1, y2, src_e, dst_e):
    n_pad, d = y1.shape
    if True:
        nhalf, half, npk = 1, n_pad, n_pad
        dst_parts = [dst_e]
    out = _make_segsum(npk, n_pad, d, nhalf)(
        y1, y2,
        src_e.reshape(_EPAD // _C, _C),
        jnp.concatenate(dst_parts).reshape(nhalf * (_EPAD // _C), _C),
        jnp.zeros((128, d), jnp.float32))
    res = []
    for yi in range(2):
        parts0, parts1 = [], []
        for h in range(nhalf):
            blk = out[(yi * nhalf + h) * 2 * npk:]
            lo = h * half
            m = min(half, n_pad - lo)
            parts0.append(blk[:m])
            parts1.append(blk[npk:npk + m])
        res.append(jnp.concatenate(parts0) if nhalf > 1 else parts0[0])
        res.append(jnp.concatenate(parts1) if nhalf > 1 else parts1[0])
    return tuple(res)


def _ref_agg(x, src, dst, valid, n):
    v = valid.astype(x.dtype)[:, None]
    msg = x[src] * v
    seg = jnp.where(valid, dst, n)
    return jax.ops.segment_sum(msg, seg, num_segments=n + 1)[:n]


def _pick_bn(n_pad):
    return n_pad // 4


def _down_body(nreal, bn, x_ref, a_ref, w_ref, b_ref, z_ref):
    i = pl.program_id(0)
    h = x_ref[...] + a_ref[...]
    z = jnp.dot(h, w_ref[...], preferred_element_type=jnp.float32) + b_ref[...]
    z = jnp.maximum(z, 0.0)
    row = lax.broadcasted_iota(jnp.int32, z.shape, 0) + i * bn
    z_ref[...] = jnp.where(row < nreal, z, 0.0)


def _down_conv(x_pad, a, W, b, nreal):
    n_pad = x_pad.shape[0]
    bn = _pick_bn(n_pad)
    b2 = jnp.reshape(b, (1, _D))
    return pl.pallas_call(
        functools.partial(_down_body, nreal, bn),
        grid=(n_pad // bn,),
        in_specs=[
            pl.BlockSpec((bn, _D), lambda i: (i, 0)),
            pl.BlockSpec((bn, _D), lambda i: (i, 0)),
            pl.BlockSpec((_D, _D), lambda i: (0, 0)),
            pl.BlockSpec((1, _D), lambda i: (0, 0)),
        ],
        out_specs=pl.BlockSpec((bn, _D), lambda i: (i, 0)),
        out_shape=jax.ShapeDtypeStruct((n_pad, _D), jnp.float32),
    )(x_pad, a, W, b2)


def _bott_body(nreal, bn, x_ref, a0_ref, a1_ref, r0_ref, r1_ref, w_ref,
               wr_ref, b_ref, z_ref):
    i = pl.program_id(0)
    h = x_ref[...] + a0_ref[...] + a1_ref[...]
    z = jnp.dot(h, w_ref[...], preferred_element_type=jnp.float32)
    rcol = (r0_ref[...] + r1_ref[...])[:, 0:1]
    z = z + rcol * wr_ref[...] + b_ref[...]
    z = jnp.maximum(z, 0.0)
    row = lax.broadcasted_iota(jnp.int32, z.shape, 0) + i * bn
    z_ref[...] = jnp.where(row < nreal, z, 0.0)


def _bott_conv(x_pad, a0, a1, r0, r1, Wb, bb, Re, nreal):
    n_pad = x_pad.shape[0]
    bn = _pick_bn(n_pad)
    W128 = Wb[:_D]
    wr = jnp.reshape(Wb[_D], (1, _D))
    b_eff = jnp.reshape(bb + Re[0] * Wb[_D], (1, _D))
    return pl.pallas_call(
        functools.partial(_bott_body, nreal, bn),
        grid=(n_pad // bn,),
        in_specs=[
            pl.BlockSpec((bn, _D), lambda i: (i, 0)),
            pl.BlockSpec((bn, _D), lambda i: (i, 0)),
            pl.BlockSpec((bn, _D), lambda i: (i, 0)),
            pl.BlockSpec((bn, _D), lambda i: (i, 0)),
            pl.BlockSpec((bn, _D), lambda i: (i, 0)),
            pl.BlockSpec((_D, _D), lambda i: (0, 0)),
            pl.BlockSpec((1, _D), lambda i: (0, 0)),
            pl.BlockSpec((1, _D), lambda i: (0, 0)),
        ],
        out_specs=pl.BlockSpec((bn, _D), lambda i: (i, 0)),
        out_shape=jax.ShapeDtypeStruct((n_pad, _D), jnp.float32),
    )(x_pad, a0, a1, r0, r1, W128, wr, b_eff)


def _up_body(nreal, bn, nout, final, xu_ref, au0_ref, au1_ref, sk_ref,
             as0_ref, as1_ref, wt_ref, wb_ref, b_ref, z_ref):
    i = pl.program_id(0)
    hu = xu_ref[...] + au0_ref[...] + au1_ref[...]
    hs = sk_ref[...] + as0_ref[...] + as1_ref[...]
    z = (jnp.dot(hu, wt_ref[...], preferred_element_type=jnp.float32)
         + jnp.dot(hs, wb_ref[...], preferred_element_type=jnp.float32)
         + b_ref[...])
    z = jnp.maximum(z, 0.0)
    if final:
        lane = lax.broadcasted_iota(jnp.int32, z.shape, 1)
        z = jnp.where(lane < nout, z, -1e30)
        m = jnp.max(z, axis=1, keepdims=True)
        y = z - m
        z_ref[...] = y - jnp.log(jnp.sum(jnp.exp(y), axis=1, keepdims=True))
    else:
        row = lax.broadcasted_iota(jnp.int32, z.shape, 0) + i * bn
        z_ref[...] = jnp.where(row < nreal, z, 0.0)


def _up_conv(xu, au0, au1, sk, as0, as1, Wu, bu, nreal, final):
    n_pad = xu.shape[0]
    bn = _pick_bn(n_pad)
    nout = Wu.shape[1]
    Wt = jnp.zeros((_D, _D), jnp.float32).at[:, :nout].set(Wu[:_D])
    Wb2 = jnp.zeros((_D, _D), jnp.float32).at[:, :nout].set(Wu[_D:])
    b2 = jnp.zeros((1, _D), jnp.float32).at[0, :nout].set(bu)
    return pl.pallas_call(
        functools.partial(_up_body, nreal, bn, nout, final),
        grid=(n_pad // bn,),
        in_specs=[pl.BlockSpec((bn, _D), lambda i: (i, 0))] * 6
        + [
            pl.BlockSpec((_D, _D), lambda i: (0, 0)),
            pl.BlockSpec((_D, _D), lambda i: (0, 0)),
            pl.BlockSpec((1, _D), lambda i: (0, 0)),
        ],
        out_specs=pl.BlockSpec((bn, _D), lambda i: (i, 0)),
        out_shape=jax.ShapeDtypeStruct((n_pad, _D), jnp.float32),
    )(xu, au0, au1, sk, as0, as1, Wt, Wb2, b2)


_NS = [10000, 5000, 2500, 1250]
_NPADS = [10112, 5120, 2560, 1280]


def _pad_edges(se, de):
    pe = _EPAD - se.shape[0]
    return (jnp.concatenate([se, jnp.full((pe,), 0, jnp.int32)]),
            jnp.concatenate([de, jnp.full((pe,), 0, jnp.int32)]))


def kernel(x, edge_index, Re, Wd0, bd0, Wd1, bd1, Wd2, bd2, p0, p1, p2,
           Wb, bb, Wu0, bu0, Wu1, bu1, Wu2, bu2):
    Wd = [Wd0, Wd1, Wd2]
    bd = [bd0, bd1, bd2]
    pp = [p0, p1, p2]
    Wu = [Wu0, Wu1, Wu2]
    bu = [bu0, bu1, bu2]

    src = edge_index[0]
    dst = edge_index[1]
    valid = jnp.ones((_E,), dtype=bool)

    x_pad = jnp.zeros((_NPADS[0], _D), jnp.float32).at[:_N].set(x)
    src_e, dst_e = _pad_edges(src, dst)
    src_e = jnp.where(jnp.arange(_EPAD) < _E, src_e, _NS[0])

    skips = []
    edge_lvls = []
    indcs = []

    for i in range(_DEPTH):
        n, n_pad = _NS[i], _NPADS[i]
        kn, kn_pad = _NS[i + 1], _NPADS[i + 1]
        agg = _ref_agg(x_pad[:n], src, dst, valid, n)
        a_pad = jnp.zeros((n_pad, _D), jnp.float32).at[:n].set(agg)
        z_pad = _down_conv(x_pad, a_pad, Wd[i], bd[i], n)
        score = (z_pad[:n] @ pp[i]) / (jnp.linalg.norm(pp[i]) + 1e-8)
        skips.append(z_pad)
        edge_lvls.append((src_e, dst_e))

        vals, idx = jax.lax.top_k(score, kn)
        gate = jnp.tanh(vals)
        indcs.append(idx)
        idx_pad = jnp.concatenate(
            [idx, jnp.zeros((kn_pad - kn,), jnp.int32)])
        gate_pad = jnp.concatenate(
            [gate, jnp.zeros((kn_pad - kn,), jnp.float32)])
        x_pad = z_pad[idx_pad] * gate_pad[:, None]
        sel = jnp.zeros((n,), dtype=bool).at[idx].set(True)
        perm = jnp.zeros((n,), jnp.int32).at[idx].set(
            jnp.arange(kn, dtype=jnp.int32))
        valid = valid & sel[src] & sel[dst]
        src = perm[src]
        dst = perm[dst]
        se = jnp.where(valid, src, kn)
        de = jnp.where(valid, dst, 0)
        src_e, dst_e = _pad_edges(se, de)
        src_e = jnp.where(jnp.arange(_EPAD) < _E, src_e, kn)

    n, n_pad = _NS[_DEPTH], _NPADS[_DEPTH]
    re_col = jnp.zeros((n_pad, _D), jnp.float32).at[:n, 0].set(Re[0])
    a0, a1, r0, r1 = _segsum_pair(x_pad, re_col, src_e, dst_e)
    x_pad = _bott_conv(x_pad, a0, a1, r0, r1, Wb, bb, Re, n)

    for i in range(_DEPTH):
        up = _DEPTH - i - 1
        n, n_pad = _NS[up], _NPADS[up]
        kn = _NS[up + 1]
        sk = skips[up]
        src_e, dst_e = edge_lvls[up]
        idx = indcs[up]
        xu = jnp.zeros((n_pad, _D), jnp.float32).at[idx].set(x_pad[:kn])
        if up == 0:
            au0 = jax.ops.segment_sum(xu[src_e], dst_e, num_segments=n_pad)
            as0 = jax.ops.segment_sum(sk[src_e], dst_e, num_segments=n_pad)
            au1 = jnp.zeros_like(au0)
            as1 = jnp.zeros_like(as0)
        else:
            au0, au1, as0, as1 = _segsum_pair(xu, sk, src_e, dst_e)
        x_pad = _up_conv(xu, au0, au1, sk, as0, as1, Wu[i], bu[i], n,
                         final=(i == _DEPTH - 1))
    return x_pad[:_N, :_NC]


# speedup vs baseline: 2.3818x; 2.3818x over previous
import functools

import jax
import jax.numpy as jnp
from jax import lax
from jax.experimental import pallas as pl
from jax.experimental.pallas import tpu as pltpu
from jax.experimental.pallas import tpu_sc as plsc

_N = 10000
_E = 320000
_D = 128
_DEPTH = 3
_NC = 102

_NCORES = 2
_NSUB = 16
_NW = _NCORES * _NSUB
_C = 128
_EPAD = 327680
_EPW = _EPAD // _NW
_NCH = _EPW // _C


@functools.cache
def _make_segsum(n_pad, yrows, d, nhalf):
    rpt = n_pad // _NSUB
    mesh = plsc.VectorSubcoreMesh(
        core_axis_name="c", subcore_axis_name="s",
        num_cores=_NCORES, num_subcores=_NSUB)

    nbuf = 4

    def body(y1_hbm, y2_hbm, srce_hbm, dste_hbm, zeros_hbm, out_hbm,
             sidx_all, didx_all, rows_v,
             agg_sh, *sems):
        gsems = sems[:nbuf]
        ssems = sems[nbuf:]
        c = lax.axis_index("c")
        s = lax.axis_index("s")
        wid = s * _NCORES + c
        pltpu.sync_copy(srce_hbm.at[pl.ds(wid * _NCH, _NCH)], sidx_all)
        for h in range(nhalf):
            pltpu.sync_copy(
                dste_hbm.at[pl.ds(h * (_EPAD // _C) + wid * _NCH, _NCH)],
                didx_all.at[pl.ds(h * _NCH, _NCH)])

        def one_pass(y_hbm, h, out_base):
            dbase = h * _NCH
            for z0 in range(0, rpt, 128):
                zw = min(128, rpt - z0)
                pltpu.sync_copy(zeros_hbm.at[pl.ds(0, zw)],
                                agg_sh.at[pl.ds(s * rpt + z0, zw)])
            plsc.subcore_barrier()
            for b in range(nbuf):
                pltpu.async_copy(y_hbm.at[sidx_all.at[b]], rows_v.at[b],
                                 gsems[b])

            def step(g, carry):
                for b in range(nbuf):
                    j = g * nbuf + b
                    pltpu.make_async_copy(y_hbm.at[sidx_all.at[j]],
                                          rows_v.at[b], gsems[b]).wait()
                    pltpu.async_copy(rows_v.at[b],
                                     agg_sh.at[didx_all.at[dbase + j]],
                                     ssems[b], add=True)
                for b in range(nbuf):
                    j = g * nbuf + b
                    jn = jnp.minimum(j + nbuf, _NCH - 1)
                    pltpu.make_async_copy(rows_v.at[b],
                                          agg_sh.at[didx_all.at[dbase + jn]],
                                          ssems[b]).wait()
                    pltpu.async_copy(y_hbm.at[sidx_all.at[jn]],
                                     rows_v.at[b], gsems[b])
                return carry

            lax.fori_loop(0, _NCH // nbuf, step, 0)
            for b in range(nbuf):
                pltpu.make_async_copy(y_hbm.at[sidx_all.at[b]],
                                      rows_v.at[b], gsems[b]).wait()
            plsc.subcore_barrier()
            pltpu.sync_copy(
                agg_sh.at[pl.ds(s * rpt, rpt)],
                out_hbm.at[pl.ds(out_base + c * n_pad + s * rpt, rpt)])
            plsc.subcore_barrier()

        for yi, y_hbm in enumerate((y1_hbm, y2_hbm)):
            for h in range(nhalf):
                one_pass(y_hbm, h, (yi * nhalf + h) * 2 * n_pad)

    return pl.kernel(
        body,
        out_type=jax.ShapeDtypeStruct((4 * nhalf * n_pad, d), jnp.float32),
        mesh=mesh,
        scratch_types=[
            pltpu.VMEM((_NCH, _C), jnp.int32),
            pltpu.VMEM((nhalf * _NCH, _C), jnp.int32),
            pltpu.VMEM((nbuf, _C, d), jnp.float32),
            pltpu.VMEM_SHARED((n_pad, d), jnp.float32),
        ] + [pltpu.SemaphoreType.DMA] * (2 * nbuf),
    )


def _segsum_pair(y1, y2, src_e, dst_e):
    n_pad, _ = y1.shape
    a0 = jax.ops.segment_sum(y1[src_e], dst_e, num_segments=n_pad)
    b0 = jax.ops.segment_sum(y2[src_e], dst_e, num_segments=n_pad)
    return a0, jnp.zeros_like(a0), b0, jnp.zeros_like(b0)


def _ref_agg(x, src, dst, valid, n):
    v = valid.astype(x.dtype)[:, None]
    msg = x[src] * v
    seg = jnp.where(valid, dst, n)
    return jax.ops.segment_sum(msg, seg, num_segments=n + 1)[:n]


def _pick_bn(n_pad):
    return n_pad // 4


def _down_body(nreal, bn, x_ref, a_ref, w_ref, b_ref, z_ref):
    i = pl.program_id(0)
    h = x_ref[...] + a_ref[...]
    z = jnp.dot(h, w_ref[...], preferred_element_type=jnp.float32) + b_ref[...]
    z = jnp.maximum(z, 0.0)
    row = lax.broadcasted_iota(jnp.int32, z.shape, 0) + i * bn
    z_ref[...] = jnp.where(row < nreal, z, 0.0)


def _down_conv(x_pad, a, W, b, nreal):
    n_pad = x_pad.shape[0]
    bn = _pick_bn(n_pad)
    b2 = jnp.reshape(b, (1, _D))
    return pl.pallas_call(
        functools.partial(_down_body, nreal, bn),
        grid=(n_pad // bn,),
        in_specs=[
            pl.BlockSpec((bn, _D), lambda i: (i, 0)),
            pl.BlockSpec((bn, _D), lambda i: (i, 0)),
            pl.BlockSpec((_D, _D), lambda i: (0, 0)),
            pl.BlockSpec((1, _D), lambda i: (0, 0)),
        ],
        out_specs=pl.BlockSpec((bn, _D), lambda i: (i, 0)),
        out_shape=jax.ShapeDtypeStruct((n_pad, _D), jnp.float32),
    )(x_pad, a, W, b2)


def _bott_body(nreal, bn, x_ref, a0_ref, a1_ref, r0_ref, r1_ref, w_ref,
               wr_ref, b_ref, z_ref):
    i = pl.program_id(0)
    h = x_ref[...] + a0_ref[...] + a1_ref[...]
    z = jnp.dot(h, w_ref[...], preferred_element_type=jnp.float32)
    rcol = (r0_ref[...] + r1_ref[...])[:, 0:1]
    z = z + rcol * wr_ref[...] + b_ref[...]
    z = jnp.maximum(z, 0.0)
    row = lax.broadcasted_iota(jnp.int32, z.shape, 0) + i * bn
    z_ref[...] = jnp.where(row < nreal, z, 0.0)


def _bott_conv(x_pad, a0, a1, r0, r1, Wb, bb, Re, nreal):
    n_pad = x_pad.shape[0]
    bn = _pick_bn(n_pad)
    W128 = Wb[:_D]
    wr = jnp.reshape(Wb[_D], (1, _D))
    b_eff = jnp.reshape(bb + Re[0] * Wb[_D], (1, _D))
    return pl.pallas_call(
        functools.partial(_bott_body, nreal, bn),
        grid=(n_pad // bn,),
        in_specs=[
            pl.BlockSpec((bn, _D), lambda i: (i, 0)),
            pl.BlockSpec((bn, _D), lambda i: (i, 0)),
            pl.BlockSpec((bn, _D), lambda i: (i, 0)),
            pl.BlockSpec((bn, _D), lambda i: (i, 0)),
            pl.BlockSpec((bn, _D), lambda i: (i, 0)),
            pl.BlockSpec((_D, _D), lambda i: (0, 0)),
            pl.BlockSpec((1, _D), lambda i: (0, 0)),
            pl.BlockSpec((1, _D), lambda i: (0, 0)),
        ],
        out_specs=pl.BlockSpec((bn, _D), lambda i: (i, 0)),
        out_shape=jax.ShapeDtypeStruct((n_pad, _D), jnp.float32),
    )(x_pad, a0, a1, r0, r1, W128, wr, b_eff)


def _up_body(nreal, bn, nout, final, xu_ref, au0_ref, au1_ref, sk_ref,
             as0_ref, as1_ref, wt_ref, wb_ref, b_ref, z_ref):
    i = pl.program_id(0)
    hu = xu_ref[...] + au0_ref[...] + au1_ref[...]
    hs = sk_ref[...] + as0_ref[...] + as1_ref[...]
    z = (jnp.dot(hu, wt_ref[...], preferred_element_type=jnp.float32)
         + jnp.dot(hs, wb_ref[...], preferred_element_type=jnp.float32)
         + b_ref[...])
    z = jnp.maximum(z, 0.0)
    if final:
        lane = lax.broadcasted_iota(jnp.int32, z.shape, 1)
        z = jnp.where(lane < nout, z, -1e30)
        m = jnp.max(z, axis=1, keepdims=True)
        y = z - m
        z_ref[...] = y - jnp.log(jnp.sum(jnp.exp(y), axis=1, keepdims=True))
    else:
        row = lax.broadcasted_iota(jnp.int32, z.shape, 0) + i * bn
        z_ref[...] = jnp.where(row < nreal, z, 0.0)


def _up_conv(xu, au0, au1, sk, as0, as1, Wu, bu, nreal, final):
    n_pad = xu.shape[0]
    bn = _pick_bn(n_pad)
    nout = Wu.shape[1]
    Wt = jnp.zeros((_D, _D), jnp.float32).at[:, :nout].set(Wu[:_D])
    Wb2 = jnp.zeros((_D, _D), jnp.float32).at[:, :nout].set(Wu[_D:])
    b2 = jnp.zeros((1, _D), jnp.float32).at[0, :nout].set(bu)
    return pl.pallas_call(
        functools.partial(_up_body, nreal, bn, nout, final),
        grid=(n_pad // bn,),
        in_specs=[pl.BlockSpec((bn, _D), lambda i: (i, 0))] * 6
        + [
            pl.BlockSpec((_D, _D), lambda i: (0, 0)),
            pl.BlockSpec((_D, _D), lambda i: (0, 0)),
            pl.BlockSpec((1, _D), lambda i: (0, 0)),
        ],
        out_specs=pl.BlockSpec((bn, _D), lambda i: (i, 0)),
        out_shape=jax.ShapeDtypeStruct((n_pad, _D), jnp.float32),
    )(xu, au0, au1, sk, as0, as1, Wt, Wb2, b2)


_NS = [10000, 5000, 2500, 1250]
_NPADS = [10112, 5120, 2560, 1280]


def _pad_edges(se, de):
    pe = _EPAD - se.shape[0]
    return (jnp.concatenate([se, jnp.full((pe,), 0, jnp.int32)]),
            jnp.concatenate([de, jnp.full((pe,), 0, jnp.int32)]))


def kernel(x, edge_index, Re, Wd0, bd0, Wd1, bd1, Wd2, bd2, p0, p1, p2,
           Wb, bb, Wu0, bu0, Wu1, bu1, Wu2, bu2):
    Wd = [Wd0, Wd1, Wd2]
    bd = [bd0, bd1, bd2]
    pp = [p0, p1, p2]
    Wu = [Wu0, Wu1, Wu2]
    bu = [bu0, bu1, bu2]

    src = edge_index[0]
    dst = edge_index[1]
    valid = jnp.ones((_E,), dtype=bool)

    x_pad = jnp.zeros((_NPADS[0], _D), jnp.float32).at[:_N].set(x)
    src_e, dst_e = _pad_edges(src, dst)
    src_e = jnp.where(jnp.arange(_EPAD) < _E, src_e, _NS[0])

    skips = []
    edge_lvls = []
    indcs = []

    for i in range(_DEPTH):
        n, n_pad = _NS[i], _NPADS[i]
        kn, kn_pad = _NS[i + 1], _NPADS[i + 1]
        agg = _ref_agg(x_pad[:n], src, dst, valid, n)
        a_pad = jnp.zeros((n_pad, _D), jnp.float32).at[:n].set(agg)
        z_pad = _down_conv(x_pad, a_pad, Wd[i], bd[i], n)
        score = (z_pad[:n] @ pp[i]) / (jnp.linalg.norm(pp[i]) + 1e-8)
        skips.append(z_pad)
        edge_lvls.append((src_e, dst_e))

        vals, idx = jax.lax.top_k(score, kn)
        gate = jnp.tanh(vals)
        indcs.append(idx)
        idx_pad = jnp.concatenate(
            [idx, jnp.zeros((kn_pad - kn,), jnp.int32)])
        gate_pad = jnp.concatenate(
            [gate, jnp.zeros((kn_pad - kn,), jnp.float32)])
        x_pad = z_pad[idx_pad] * gate_pad[:, None]
        sel = jnp.zeros((n,), dtype=bool).at[idx].set(True)
        perm = jnp.zeros((n,), jnp.int32).at[idx].set(
            jnp.arange(kn, dtype=jnp.int32))
        valid = valid & sel[src] & sel[dst]
        src = perm[src]
        dst = perm[dst]
        se = jnp.where(valid, src, kn)
        de = jnp.where(valid, dst, 0)
        src_e, dst_e = _pad_edges(se, de)
        src_e = jnp.where(jnp.arange(_EPAD) < _E, src_e, kn)

    n, n_pad = _NS[_DEPTH], _NPADS[_DEPTH]
    re_col = jnp.zeros((n_pad, _D), jnp.float32).at[:n, 0].set(Re[0])
    a0, a1, r0, r1 = _segsum_pair(x_pad, re_col, src_e, dst_e)
    x_pad = _bott_conv(x_pad, a0, a1, r0, r1, Wb, bb, Re, n)

    for i in range(_DEPTH):
        up = _DEPTH - i - 1
        n, n_pad = _NS[up], _NPADS[up]
        kn = _NS[up + 1]
        sk = skips[up]
        src_e, dst_e = edge_lvls[up]
        idx = indcs[up]
        xu = jnp.zeros((n_pad, _D), jnp.float32).at[idx].set(x_pad[:kn])
        au0, au1, as0, as1 = _segsum_pair(xu, sk, src_e, dst_e)
        x_pad = _up_conv(xu, au0, au1, sk, as0, as1, Wu[i], bu[i], n,
                         final=(i == _DEPTH - 1))
    return x_pad[:_N, :_NC]


# speedup vs baseline: 2.4476x; 1.0276x over previous
import functools

import jax
import jax.numpy as jnp
from jax import lax
from jax.experimental import pallas as pl
from jax.experimental.pallas import tpu as pltpu
from jax.experimental.pallas import tpu_sc as plsc

_N = 10000
_E = 320000
_D = 128
_DEPTH = 3
_NC = 102

_NCORES = 2
_NSUB = 16
_NW = _NCORES * _NSUB
_C = 128
_EPAD = 327680
_EPW = _EPAD // _NW
_NCH = _EPW // _C


@functools.cache
def _make_segsum(n_pad, yrows, d, nhalf):
    rpt = n_pad // _NSUB
    mesh = plsc.VectorSubcoreMesh(
        core_axis_name="c", subcore_axis_name="s",
        num_cores=_NCORES, num_subcores=_NSUB)

    nbuf = 4

    def body(y1_hbm, y2_hbm, srce_hbm, dste_hbm, zeros_hbm, out_hbm,
             sidx_all, didx_all, rows_v,
             agg_sh, *sems):
        gsems = sems[:nbuf]
        ssems = sems[nbuf:]
        c = lax.axis_index("c")
        s = lax.axis_index("s")
        wid = s * _NCORES + c
        pltpu.sync_copy(srce_hbm.at[pl.ds(wid * _NCH, _NCH)], sidx_all)
        for h in range(nhalf):
            pltpu.sync_copy(
                dste_hbm.at[pl.ds(h * (_EPAD // _C) + wid * _NCH, _NCH)],
                didx_all.at[pl.ds(h * _NCH, _NCH)])

        def one_pass(y_hbm, h, out_base):
            dbase = h * _NCH
            for z0 in range(0, rpt, 128):
                zw = min(128, rpt - z0)
                pltpu.sync_copy(zeros_hbm.at[pl.ds(0, zw)],
                                agg_sh.at[pl.ds(s * rpt + z0, zw)])
            plsc.subcore_barrier()
            for b in range(nbuf):
                pltpu.async_copy(y_hbm.at[sidx_all.at[b]], rows_v.at[b],
                                 gsems[b])

            def step(g, carry):
                for b in range(nbuf):
                    j = g * nbuf + b
                    pltpu.make_async_copy(y_hbm.at[sidx_all.at[j]],
                                          rows_v.at[b], gsems[b]).wait()
                    pltpu.async_copy(rows_v.at[b],
                                     agg_sh.at[didx_all.at[dbase + j]],
                                     ssems[b], add=True)
                for b in range(nbuf):
                    j = g * nbuf + b
                    jn = jnp.minimum(j + nbuf, _NCH - 1)
                    pltpu.make_async_copy(rows_v.at[b],
                                          agg_sh.at[didx_all.at[dbase + jn]],
                                          ssems[b]).wait()
                    pltpu.async_copy(y_hbm.at[sidx_all.at[jn]],
                                     rows_v.at[b], gsems[b])
                return carry

            lax.fori_loop(0, _NCH // nbuf, step, 0)
            for b in range(nbuf):
                pltpu.make_async_copy(y_hbm.at[sidx_all.at[b]],
                                      rows_v.at[b], gsems[b]).wait()
            plsc.subcore_barrier()
            pltpu.sync_copy(
                agg_sh.at[pl.ds(s * rpt, rpt)],
                out_hbm.at[pl.ds(out_base + c * n_pad + s * rpt, rpt)])
            plsc.subcore_barrier()

        for yi, y_hbm in enumerate((y1_hbm, y2_hbm)):
            for h in range(nhalf):
                one_pass(y_hbm, h, (yi * nhalf + h) * 2 * n_pad)

    return pl.kernel(
        body,
        out_type=jax.ShapeDtypeStruct((4 * nhalf * n_pad, d), jnp.float32),
        mesh=mesh,
        scratch_types=[
            pltpu.VMEM((_NCH, _C), jnp.int32),
            pltpu.VMEM((nhalf * _NCH, _C), jnp.int32),
            pltpu.VMEM((nbuf, _C, d), jnp.float32),
            pltpu.VMEM_SHARED((n_pad, d), jnp.float32),
        ] + [pltpu.SemaphoreType.DMA] * (2 * nbuf),
    )


def _segsum_pair(y1, y2, src_e, dst_e):
    n_pad, _ = y1.shape
    a0 = jax.ops.segment_sum(y1[src_e], dst_e, num_segments=n_pad)
    b0 = jax.ops.segment_sum(y2[src_e], dst_e, num_segments=n_pad)
    return a0, jnp.zeros_like(a0), b0, jnp.zeros_like(b0)


def _ref_agg(x, src, dst, valid, n):
    v = valid.astype(x.dtype)[:, None]
    msg = x[src] * v
    seg = jnp.where(valid, dst, n)
    return jax.ops.segment_sum(msg, seg, num_segments=n + 1)[:n]


def _pick_bn(n_pad):
    bn = n_pad // 4
    return bn if bn % 8 == 0 else n_pad


def _down_body(nreal, bn, x_ref, a_ref, w_ref, b_ref, z_ref):
    i = pl.program_id(0)
    h = x_ref[...] + a_ref[...]
    z = jnp.dot(h, w_ref[...], preferred_element_type=jnp.float32) + b_ref[...]
    z = jnp.maximum(z, 0.0)
    row = lax.broadcasted_iota(jnp.int32, z.shape, 0) + i * bn
    z_ref[...] = jnp.where(row < nreal, z, 0.0)


def _down_conv(x_pad, a, W, b, nreal):
    n_pad = x_pad.shape[0]
    bn = _pick_bn(n_pad)
    b2 = jnp.reshape(b, (1, _D))
    return pl.pallas_call(
        functools.partial(_down_body, nreal, bn),
        grid=(n_pad // bn,),
        in_specs=[
            pl.BlockSpec((bn, _D), lambda i: (i, 0)),
            pl.BlockSpec((bn, _D), lambda i: (i, 0)),
            pl.BlockSpec((_D, _D), lambda i: (0, 0)),
            pl.BlockSpec((1, _D), lambda i: (0, 0)),
        ],
        out_specs=pl.BlockSpec((bn, _D), lambda i: (i, 0)),
        out_shape=jax.ShapeDtypeStruct((n_pad, _D), jnp.float32),
    )(x_pad, a, W, b2)


def _bott_body(nreal, bn, x_ref, a0_ref, a1_ref, r0_ref, r1_ref, w_ref,
               wr_ref, b_ref, z_ref):
    i = pl.program_id(0)
    h = x_ref[...] + a0_ref[...] + a1_ref[...]
    z = jnp.dot(h, w_ref[...], preferred_element_type=jnp.float32)
    rcol = (r0_ref[...] + r1_ref[...])[:, 0:1]
    z = z + rcol * wr_ref[...] + b_ref[...]
    z = jnp.maximum(z, 0.0)
    row = lax.broadcasted_iota(jnp.int32, z.shape, 0) + i * bn
    z_ref[...] = jnp.where(row < nreal, z, 0.0)


def _bott_conv(x_pad, a0, a1, r0, r1, Wb, bb, Re, nreal):
    n_pad = x_pad.shape[0]
    bn = _pick_bn(n_pad)
    W128 = Wb[:_D]
    wr = jnp.reshape(Wb[_D], (1, _D))
    b_eff = jnp.reshape(bb + Re[0] * Wb[_D], (1, _D))
    return pl.pallas_call(
        functools.partial(_bott_body, nreal, bn),
        grid=(n_pad // bn,),
        in_specs=[
            pl.BlockSpec((bn, _D), lambda i: (i, 0)),
            pl.BlockSpec((bn, _D), lambda i: (i, 0)),
            pl.BlockSpec((bn, _D), lambda i: (i, 0)),
            pl.BlockSpec((bn, _D), lambda i: (i, 0)),
            pl.BlockSpec((bn, _D), lambda i: (i, 0)),
            pl.BlockSpec((_D, _D), lambda i: (0, 0)),
            pl.BlockSpec((1, _D), lambda i: (0, 0)),
            pl.BlockSpec((1, _D), lambda i: (0, 0)),
        ],
        out_specs=pl.BlockSpec((bn, _D), lambda i: (i, 0)),
        out_shape=jax.ShapeDtypeStruct((n_pad, _D), jnp.float32),
    )(x_pad, a0, a1, r0, r1, W128, wr, b_eff)


def _up_body(nreal, bn, nout, final, xu_ref, au0_ref, au1_ref, sk_ref,
             as0_ref, as1_ref, wt_ref, wb_ref, b_ref, z_ref):
    i = pl.program_id(0)
    hu = xu_ref[...] + au0_ref[...] + au1_ref[...]
    hs = sk_ref[...] + as0_ref[...] + as1_ref[...]
    z = (jnp.dot(hu, wt_ref[...], preferred_element_type=jnp.float32)
         + jnp.dot(hs, wb_ref[...], preferred_element_type=jnp.float32)
         + b_ref[...])
    z = jnp.maximum(z, 0.0)
    if final:
        lane = lax.broadcasted_iota(jnp.int32, z.shape, 1)
        z = jnp.where(lane < nout, z, -1e30)
        m = jnp.max(z, axis=1, keepdims=True)
        y = z - m
        z_ref[...] = y - jnp.log(jnp.sum(jnp.exp(y), axis=1, keepdims=True))
    else:
        row = lax.broadcasted_iota(jnp.int32, z.shape, 0) + i * bn
        z_ref[...] = jnp.where(row < nreal, z, 0.0)


def _up_conv(xu, au0, au1, sk, as0, as1, Wu, bu, nreal, final):
    n_pad = xu.shape[0]
    bn = _pick_bn(n_pad)
    nout = Wu.shape[1]
    Wt = jnp.zeros((_D, _D), jnp.float32).at[:, :nout].set(Wu[:_D])
    Wb2 = jnp.zeros((_D, _D), jnp.float32).at[:, :nout].set(Wu[_D:])
    b2 = jnp.zeros((1, _D), jnp.float32).at[0, :nout].set(bu)
    return pl.pallas_call(
        functools.partial(_up_body, nreal, bn, nout, final),
        grid=(n_pad // bn,),
        in_specs=[pl.BlockSpec((bn, _D), lambda i: (i, 0))] * 6
        + [
            pl.BlockSpec((_D, _D), lambda i: (0, 0)),
            pl.BlockSpec((_D, _D), lambda i: (0, 0)),
            pl.BlockSpec((1, _D), lambda i: (0, 0)),
        ],
        out_specs=pl.BlockSpec((bn, _D), lambda i: (i, 0)),
        out_shape=jax.ShapeDtypeStruct((n_pad, _D), jnp.float32),
    )(xu, au0, au1, sk, as0, as1, Wt, Wb2, b2)


_NS = [10000, 5000, 2500, 1250]
_NPADS = [10000, 5000, 2500, 1250]


def _pad_edges(se, de):
    pe = _EPAD - se.shape[0]
    return (jnp.concatenate([se, jnp.full((pe,), 0, jnp.int32)]),
            jnp.concatenate([de, jnp.full((pe,), 0, jnp.int32)]))


def kernel(x, edge_index, Re, Wd0, bd0, Wd1, bd1, Wd2, bd2, p0, p1, p2,
           Wb, bb, Wu0, bu0, Wu1, bu1, Wu2, bu2):
    Wd = [Wd0, Wd1, Wd2]
    bd = [bd0, bd1, bd2]
    pp = [p0, p1, p2]
    Wu = [Wu0, Wu1, Wu2]
    bu = [bu0, bu1, bu2]

    src = edge_index[0]
    dst = edge_index[1]
    valid = jnp.ones((_E,), dtype=bool)

    skips = []
    edge_lvls = []
    indcs = []

    for i in range(_DEPTH):
        n = _NS[i]
        kn = _NS[i + 1]
        agg = _ref_agg(x, src, dst, valid, n)
        z = _down_conv(x, agg, Wd[i], bd[i], n)
        score = (z @ pp[i]) / (jnp.linalg.norm(pp[i]) + 1e-8)
        skips.append(z)
        edge_lvls.append((src, dst, valid))

        vals, idx = jax.lax.top_k(score, kn)
        gate = jnp.tanh(vals)
        indcs.append(idx)
        x = z[idx] * gate[:, None]
        sel = jnp.zeros((n,), dtype=bool).at[idx].set(True)
        perm = jnp.zeros((n,), jnp.int32).at[idx].set(
            jnp.arange(kn, dtype=jnp.int32))
        valid = valid & sel[src] & sel[dst]
        src = perm[src]
        dst = perm[dst]

    n = _NS[_DEPTH]
    agg = _ref_agg(x, src, dst, valid, n)
    ones = jnp.full((n, 1), Re[0], jnp.float32)
    r = _ref_agg(ones, src, dst, valid, n)
    zc = jnp.zeros_like(agg)
    x = _bott_conv(x, agg, zc, jnp.broadcast_to(r, (n, _D)), zc,
                   Wb, bb, Re, n)

    for i in range(_DEPTH):
        up = _DEPTH - i - 1
        n = _NS[up]
        kn = _NS[up + 1]
        sk = skips[up]
        src, dst, valid = edge_lvls[up]
        idx = indcs[up]
        xu = jnp.zeros((n, _D), jnp.float32).at[idx].set(x)
        au = _ref_agg(xu, src, dst, valid, n)
        as_ = _ref_agg(sk, src, dst, valid, n)
        z0 = jnp.zeros_like(au)
        x = _up_conv(xu, au, z0, sk, as_, z0, Wu[i], bu[i], n,
                     final=(i == _DEPTH - 1))
    return x[:_N, :_NC]
